# Initial kernel scaffold; baseline (speedup 1.0000x reference)
import jax
import jax.numpy as jnp
from jax.experimental import pallas as pl


def kernel(x, emb, W, b):
    raise NotImplementedError("write your pallas kernel here")


# speedup vs baseline: 12.0021x; 12.0021x over previous
import functools

import jax
import jax.numpy as jnp
from jax import lax
from jax.experimental import pallas as pl
from jax.experimental.pallas import tpu as pltpu
from jax.experimental.pallas import tpu_sc as plsc

VOCAB = 100000
EMB = 128
NCLS = 1000
BATCH = 4096
SEQ = 200

NC, NS = 2, 16
NW = NC * NS
SPW = BATCH // NW
LANES = 16
NSEG = EMB // LANES
G0 = 128
G1 = SEQ - G0


def _sc_pooled(x_flat, emb):
  mesh = plsc.VectorSubcoreMesh(
      core_axis_name="c", subcore_axis_name="s",
      num_cores=NC, num_subcores=NS)

  @functools.partial(
      pl.kernel,
      out_type=jax.ShapeDtypeStruct((BATCH, EMB), jnp.float32),
      mesh=mesh,
      scratch_types=[
          pltpu.VMEM((SPW * SEQ,), jnp.int32),
          pltpu.VMEM((2, SEQ, EMB), jnp.float32),
          pltpu.VMEM((SPW, EMB), jnp.float32),
          pltpu.SemaphoreType.DMA,
          pltpu.SemaphoreType.DMA,
      ],
  )
  def k(x_hbm, emb_hbm, out_hbm, idx_v, rows_v, out_v, sem0, sem1):
    wid = lax.axis_index("s") * NC + lax.axis_index("c")
    ibase = wid * (SPW * SEQ)
    pltpu.sync_copy(x_hbm.at[pl.ds(ibase, SPW * SEQ)], idx_v)

    sems = (sem0, sem1)

    def issue(s, slot, sem):
      off = s * SEQ
      pltpu.async_copy(emb_hbm.at[idx_v.at[pl.ds(off, G0)]],
                       rows_v.at[slot].at[pl.ds(0, G0)], sem)
      pltpu.async_copy(emb_hbm.at[idx_v.at[pl.ds(off + G0, G1)]],
                       rows_v.at[slot].at[pl.ds(G0, G1)], sem)

    def wait(slot, sem):
      pltpu.make_async_copy(emb_hbm.at[idx_v.at[pl.ds(0, G0)]],
                            rows_v.at[slot].at[pl.ds(0, G0)], sem).wait()
      pltpu.make_async_copy(emb_hbm.at[idx_v.at[pl.ds(0, G1)]],
                            rows_v.at[slot].at[pl.ds(G0, G1)], sem).wait()

    def accumulate(s, slot):
      def rbody(r, acc):
        return tuple(acc[j] + rows_v[slot, r, pl.ds(j * LANES, LANES)]
                     for j in range(NSEG))
      zero = tuple(jnp.zeros((LANES,), jnp.float32) for _ in range(NSEG))
      acc = lax.fori_loop(0, SEQ, rbody, zero)
      for j in range(NSEG):
        out_v[s, pl.ds(j * LANES, LANES)] = acc[j] * (1.0 / SEQ)

    issue(0, 0, sem0)

    def sample_body(s, carry):
      for p in (0, 1):
        @pl.when(lax.rem(s, 2) == p)
        def _():
          @pl.when(s + 1 < SPW)
          def _():
            issue(s + 1, 1 - p, sems[1 - p])
          wait(p, sems[p])
          accumulate(s, p)
      return carry

    lax.fori_loop(0, SPW, sample_body, 0)
    pltpu.sync_copy(out_v, out_hbm.at[pl.ds(wid * SPW, SPW)])

  return k(x_flat, emb)


def _tc_head(pooled, W, b2d):
  BB = 512

  def body(p_ref, w_ref, b_ref, o_ref):
    x = p_ref[...].astype(jnp.bfloat16)
    w = w_ref[...].astype(jnp.bfloat16)
    z = lax.dot_general(x, w, (((1,), (1,)), ((), ())),
                        preferred_element_type=jnp.float32)
    z = z + b_ref[...]
    m = jnp.max(z, axis=1, keepdims=True)
    e = jnp.exp(z - m)
    lse = jnp.log(jnp.sum(e, axis=1, keepdims=True)) + m
    o_ref[...] = z - lse

  return pl.pallas_call(
      body,
      grid=(BATCH // BB,),
      in_specs=[
          pl.BlockSpec((BB, EMB), lambda i: (i, 0)),
          pl.BlockSpec((NCLS, EMB), lambda i: (0, 0)),
          pl.BlockSpec((1, NCLS), lambda i: (0, 0)),
      ],
      out_specs=pl.BlockSpec((BB, NCLS), lambda i: (i, 0)),
      out_shape=jax.ShapeDtypeStruct((BATCH, NCLS), jnp.float32),
  )(pooled, W, b2d)


def kernel(x, emb, W, b):
  x_flat = x.reshape(-1).astype(jnp.int32)
  pooled = _sc_pooled(x_flat, emb)
  return _tc_head(pooled, W, b.reshape(1, NCLS))
